# baseline (speedup 1.0000x reference)
import functools

import jax
import jax.numpy as jnp
import numpy as np
from jax import lax
from jax.experimental import pallas as pl
from jax.experimental.pallas import tpu as pltpu
from jax.experimental.pallas import tpu_sc as plsc

TOP_K = 8
NUM_EXPERTS = 64
NUM_TOKENS_TOTAL = 32768
NC, NS, L = 2, 16, 16
NW = NC * NS
TOK_PER_W = NUM_TOKENS_TOTAL // NW
PAIRS_PER_W = TOK_PER_W // 2
CHUNK = 512

_GDN = lax.GatherDimensionNumbers(
    offset_dims=(), collapsed_slice_dims=(0,), start_index_map=(0,)
)


def _perm(x, idx):
    return lax.gather(
        x,
        idx[:, None],
        _GDN,
        (1,),
        mode=lax.GatherScatterMode.PROMISE_IN_BOUNDS,
    )


def _rev(x):
    return lax.rev(x, (0,))


def _merge(ka, va, kb, vb):
    krb = _rev(kb)
    vrb = _rev(vb)
    c = ka >= krb
    return jnp.where(c, ka, krb), jnp.where(c, va, vrb)


def _topk_body(x_hbm, w_hbm, id_hbm, x_v, w_v, id_v):
    wid = lax.axis_index("c") * NS + lax.axis_index("s")
    base = wid * TOK_PER_W

    lane = lax.iota(jnp.int32, L)
    iotas = [lane + j * L for j in range(4)]
    lt8 = lane < TOP_K
    zeros16 = lane * 0
    fifteens = zeros16 + 15
    pack_hi = jnp.maximum(lane - TOP_K, 0)

    m_idx = jnp.where(lt8, 0, TOP_K)
    sevens = zeros16 + 7

    def one_token(t):
        ks, vs = [], []
        for j in range(4):
            k, v = plsc.sort_key_val(
                x_v[t, pl.ds(j * L, L)], iotas[j], descending=True
            )
            ks.append(k)
            vs.append(v)
        hk1, hv1 = _merge(ks[0], vs[0], ks[1], vs[1])
        hk2, hv2 = _merge(ks[2], vs[2], ks[3], vs[3])
        sk1, sv1 = plsc.sort_key_val(hk1, hv1, descending=True)
        sk2, sv2 = plsc.sort_key_val(hk2, hv2, descending=True)
        mk, mv = _merge(sk1, sv1, sk2, sv2)
        fk, fv = plsc.sort_key_val(mk, mv, descending=True)
        return fk, fv

    def make_body(c):
        def body(p):
            t0 = 2 * p
            k0, i0 = one_token(t0)
            k1, i1 = one_token(t0 + 1)
            kp = jnp.where(lt8, k0, _perm(k1, pack_hi))
            ip = jnp.where(lt8, i0, _perm(i1, pack_hi))
            mp = _perm(kp, m_idx)
            e = jnp.exp(kp - mp)
            cs = lax.cumsum(e, axis=0)
            s0 = _perm(cs, sevens)
            s1 = _perm(cs, fifteens) - s0
            s = jnp.where(lt8, s0, s1)
            q = c * (CHUNK // 2) + p
            w_v[pl.ds(q * L, L)] = e / s
            id_v[pl.ds(q * L, L)] = ip

        return body

    for c in range(TOK_PER_W // CHUNK):
        pltpu.sync_copy(x_hbm.at[pl.ds(base + c * CHUNK, CHUNK)], x_v)
        plsc.parallel_loop(0, CHUNK // 2, unroll=4)(make_body(c))

    pltpu.sync_copy(w_v, w_hbm.at[pl.ds(base * TOP_K, TOK_PER_W * TOP_K)])
    pltpu.sync_copy(id_v, id_hbm.at[pl.ds(base * TOP_K, TOK_PER_W * TOP_K)])


@jax.jit
def _sc_topk(router_logits_fp32):
    n = NUM_TOKENS_TOTAL
    mesh = plsc.VectorSubcoreMesh(core_axis_name="c", subcore_axis_name="s")
    w_flat, id_flat = pl.kernel(
        _topk_body,
        out_type=[
            jax.ShapeDtypeStruct((n * TOP_K,), jnp.float32),
            jax.ShapeDtypeStruct((n * TOP_K,), jnp.int32),
        ],
        mesh=mesh,
        compiler_params=pltpu.CompilerParams(needs_layout_passes=False),
        scratch_types=[
            pltpu.VMEM((CHUNK, NUM_EXPERTS), jnp.float32),
            pltpu.VMEM((TOK_PER_W * TOP_K,), jnp.float32),
            pltpu.VMEM((TOK_PER_W * TOP_K,), jnp.int32),
        ],
    )(router_logits_fp32)
    return w_flat, id_flat


def kernel(router_logits_fp32, topk_ids, topk_weights):
    del topk_ids, topk_weights
    w_flat, id_flat = _sc_topk(router_logits_fp32)
    w = w_flat.reshape(NUM_TOKENS_TOTAL, TOP_K)
    ids = id_flat.reshape(NUM_TOKENS_TOTAL, TOP_K)
    return (w, ids, ids)


# speedup vs baseline: 1.0833x; 1.0833x over previous
import functools

import jax
import jax.numpy as jnp
import numpy as np
from jax import lax
from jax.experimental import pallas as pl
from jax.experimental.pallas import tpu as pltpu
from jax.experimental.pallas import tpu_sc as plsc

TOP_K = 8
NUM_EXPERTS = 64
NUM_TOKENS_TOTAL = 32768
NC, NS, L = 2, 16, 16
NW = NC * NS
TOK_PER_W = NUM_TOKENS_TOTAL // NW
CHUNK = 512

_GDN = lax.GatherDimensionNumbers(
    offset_dims=(), collapsed_slice_dims=(0,), start_index_map=(0,)
)


def _perm(x, idx):
    return lax.gather(
        x, idx[:, None], _GDN, (1,), mode=lax.GatherScatterMode.PROMISE_IN_BOUNDS
    )


def _rev(x):
    return lax.rev(x, (0,))


def _merge(ka, va, kb, vb):
    krb = _rev(kb)
    vrb = _rev(vb)
    c = ka >= krb
    return jnp.where(c, ka, krb), jnp.where(c, va, vrb)


def _topk_body(x_hbm, w_hbm, id_hbm, x_v, w_v, id_v):
    wid = lax.axis_index("c") * NS + lax.axis_index("s")
    base = wid * TOK_PER_W

    lane = lax.iota(jnp.int32, L)
    iotas = [lane + j * L for j in range(4)]
    lt8 = lane < TOP_K
    zeros16 = lane * 0
    fifteens = zeros16 + 15
    pack_hi = jnp.maximum(lane - TOP_K, 0)
    m_idx = jnp.where(lt8, 0, TOP_K)
    sevens = zeros16 + 7

    def one_token(t):
        ks, vs = [], []
        for j in range(4):
            k, v = plsc.sort_key_val(
                x_v[t, pl.ds(j * L, L)], iotas[j], descending=True
            )
            ks.append(k)
            vs.append(v)
        hk1, hv1 = _merge(ks[0], vs[0], ks[1], vs[1])
        hk2, hv2 = _merge(ks[2], vs[2], ks[3], vs[3])
        sk1, sv1 = plsc.sort_key_val(hk1, hv1, descending=True)
        sk2, sv2 = plsc.sort_key_val(hk2, hv2, descending=True)
        mk, mv = _merge(sk1, sv1, sk2, sv2)
        fk, fv = plsc.sort_key_val(mk, mv, descending=True)
        return fk, fv

    def make_body(c):
        def body(p):
            t0 = 2 * p
            k0, i0 = one_token(t0)
            k1, i1 = one_token(t0 + 1)
            kp = jnp.where(lt8, k0, _perm(k1, pack_hi))
            ip = jnp.where(lt8, i0, _perm(i1, pack_hi))
            mp = _perm(kp, m_idx)
            e = jnp.exp(kp - mp)
            cs = lax.cumsum(e, axis=0)
            s0 = _perm(cs, sevens)
            s1 = _perm(cs, fifteens) - s0
            s = jnp.where(lt8, s0, s1)
            q = c * (CHUNK // 2) + p
            cidx = lane % TOP_K
            tidx = jnp.where(lt8, 2 * q, 2 * q + 1)
            plsc.store_scatter(w_v, [cidx, tidx], e / s)
            plsc.store_scatter(id_v, [cidx, tidx], ip)

        return body

    for c in range(TOK_PER_W // CHUNK):
        pltpu.sync_copy(x_hbm.at[pl.ds(base + c * CHUNK, CHUNK)], x_v)
        plsc.parallel_loop(0, CHUNK // 2, unroll=2)(make_body(c))

    pltpu.sync_copy(w_v, w_hbm.at[:, pl.ds(base, TOK_PER_W)])
    pltpu.sync_copy(id_v, id_hbm.at[:, pl.ds(base, TOK_PER_W)])


def _expand_kernel(wf_ref, if_ref, w_ref, id_ref):
    w_ref[:, :] = wf_ref[:, :].T
    id_ref[:, :] = if_ref[:, :].T


_EXP_BLOCK = 4096


def _expand(w_flat, id_flat):
    n = NUM_TOKENS_TOTAL
    grid = (n // _EXP_BLOCK,)
    return pl.pallas_call(
        _expand_kernel,
        grid=grid,
        in_specs=[
            pl.BlockSpec((TOP_K, _EXP_BLOCK), lambda i: (0, i)),
            pl.BlockSpec((TOP_K, _EXP_BLOCK), lambda i: (0, i)),
        ],
        out_specs=[
            pl.BlockSpec((_EXP_BLOCK, TOP_K), lambda i: (i, 0)),
            pl.BlockSpec((_EXP_BLOCK, TOP_K), lambda i: (i, 0)),
        ],
        out_shape=[
            jax.ShapeDtypeStruct((n, TOP_K), jnp.float32),
            jax.ShapeDtypeStruct((n, TOP_K), jnp.int32),
        ],
        compiler_params=pltpu.CompilerParams(
            dimension_semantics=("parallel",),
        ),
    )(w_flat, id_flat)


@jax.jit
def _sc_topk(router_logits_fp32):
    n = NUM_TOKENS_TOTAL
    mesh = plsc.VectorSubcoreMesh(core_axis_name="c", subcore_axis_name="s")
    w_flat, id_flat = pl.kernel(
        _topk_body,
        out_type=[
            jax.ShapeDtypeStruct((TOP_K, n), jnp.float32),
            jax.ShapeDtypeStruct((TOP_K, n), jnp.int32),
        ],
        mesh=mesh,
        compiler_params=pltpu.CompilerParams(needs_layout_passes=False),
        scratch_types=[
            pltpu.VMEM((CHUNK, NUM_EXPERTS), jnp.float32),
            pltpu.VMEM((TOP_K, TOK_PER_W), jnp.float32),
            pltpu.VMEM((TOP_K, TOK_PER_W), jnp.int32),
        ],
    )(router_logits_fp32)
    return _expand(w_flat, id_flat)


def kernel(router_logits_fp32, topk_ids, topk_weights):
    del topk_ids, topk_weights
    w, ids = _sc_topk(router_logits_fp32)
    return (w, ids, ids)
